# initial kernel scaffold (unmeasured)
import jax
import jax.numpy as jnp
from jax import lax
from jax.experimental import pallas as pl
from jax.experimental.pallas import tpu as pltpu

N_DEV = 8
_GELU_C = 0.7978845608028654


def _ring(q):
    return jnp.where(q < 4, q, 11 - q)


def _gelu(y):
    return 0.5 * y * (1.0 + jnp.tanh(_GELU_C * (y + 0.044715 * y * y * y)))


def kernel(x, w_mat):
    m_per, k = x.shape
    _, n_per = w_mat.shape
    half = m_per // 2
    m_glob = N_DEV * m_per

    def body(x_ref, w_ref, out_ref, gx_cw, gx_ccw, stage,
             send_cw, recv_cw, send_ccw, recv_ccw, local_sem):
        me = lax.axis_index("i")
        pos = _ring(me)
        right = _ring((pos + 1) % N_DEV)
        left = _ring((pos - 1) % N_DEV)

        barrier = pltpu.get_barrier_semaphore()
        for nbr in (left, right):
            pl.semaphore_signal(barrier, inc=1, device_id=(nbr,),
                                device_id_type=pl.DeviceIdType.MESH)
        pl.semaphore_wait(barrier, 2)

        def make_hop(h):
            o_send_cw = _ring((pos - h) % N_DEV)
            o_recv_cw = _ring((pos - 1 - h) % N_DEV)
            o_send_ccw = _ring((pos + h) % N_DEV)
            o_recv_ccw = _ring((pos + 1 + h) % N_DEV)
            src_cw = (x_ref.at[pl.ds(0, half)] if h == 0
                      else gx_cw.at[o_send_cw])
            src_ccw = (x_ref.at[pl.ds(half, half)] if h == 0
                       else gx_ccw.at[o_send_ccw])
            cw = pltpu.make_async_remote_copy(
                src_ref=src_cw, dst_ref=gx_cw.at[o_send_cw],
                send_sem=send_cw.at[h], recv_sem=recv_cw.at[h],
                device_id=(right,), device_id_type=pl.DeviceIdType.MESH)
            ccw = pltpu.make_async_remote_copy(
                src_ref=src_ccw, dst_ref=gx_ccw.at[o_send_ccw],
                send_sem=send_ccw.at[h], recv_sem=recv_ccw.at[h],
                device_id=(left,), device_id_type=pl.DeviceIdType.MESH)
            return cw, ccw, o_recv_cw, o_recv_ccw

        hops = [make_hop(h) for h in range(N_DEV - 1)]

        def gemm_rows(src_slice, out_row):
            cp = pltpu.make_async_copy(src_slice, stage, local_sem)
            cp.start()
            cp.wait()
            y = jnp.dot(stage[...], w_ref[...],
                        preferred_element_type=jnp.float32)
            out_ref[pl.ds(out_row, half), :] = _gelu(y)

        hops[0][0].start()
        hops[0][1].start()
        gemm_rows(x_ref.at[pl.ds(0, half)], me * m_per)
        gemm_rows(x_ref.at[pl.ds(half, half)], me * m_per + half)

        for h in range(N_DEV - 1):
            cw, ccw, o_recv_cw, o_recv_ccw = hops[h]
            cw.wait_recv()
            ccw.wait_recv()
            if h + 1 < N_DEV - 1:
                hops[h + 1][0].start()
                hops[h + 1][1].start()
            gemm_rows(gx_cw.at[o_recv_cw], o_recv_cw * m_per)
            gemm_rows(gx_ccw.at[o_recv_ccw], o_recv_ccw * m_per + half)

        for cw, ccw, _, _ in hops:
            cw.wait_send()
            ccw.wait_send()

    return pl.pallas_call(
        body,
        out_shape=jax.ShapeDtypeStruct((m_glob, n_per), jnp.float32),
        in_specs=[
            pl.BlockSpec(memory_space=pltpu.MemorySpace.HBM),
            pl.BlockSpec(memory_space=pltpu.MemorySpace.VMEM),
        ],
        out_specs=pl.BlockSpec(memory_space=pltpu.MemorySpace.VMEM),
        scratch_shapes=[
            pltpu.MemorySpace.HBM((N_DEV, half, k), jnp.float32),
            pltpu.MemorySpace.HBM((N_DEV, half, k), jnp.float32),
            pltpu.VMEM((half, k), jnp.float32),
            pltpu.SemaphoreType.DMA((N_DEV - 1,)),
            pltpu.SemaphoreType.DMA((N_DEV - 1,)),
            pltpu.SemaphoreType.DMA((N_DEV - 1,)),
            pltpu.SemaphoreType.DMA((N_DEV - 1,)),
            pltpu.SemaphoreType.DMA,
        ],
        compiler_params=pltpu.CompilerParams(
            collective_id=0,
            vmem_limit_bytes=64 * 1024 * 1024,
        ),
    )(x, w_mat)


# baseline (device time: 1326862 ns/iter reference)
import jax
import jax.numpy as jnp
from jax import lax
from jax.experimental import pallas as pl
from jax.experimental.pallas import tpu as pltpu

N_DEV = 8
_GELU_C = 0.7978845608028654


def _ring(q):
    return jnp.where(q < 4, q, 11 - q)


def _gelu(y):
    return 0.5 * y * (1.0 + jnp.tanh(_GELU_C * (y + 0.044715 * y * y * y)))


def kernel(x, w_mat):
    m_per, k = x.shape
    _, n_per = w_mat.shape
    half = m_per // 2
    m_glob = N_DEV * m_per

    def body(x_ref, w_ref, out_ref, gx_cw, gx_ccw, stage,
             send_cw, recv_cw, send_ccw, recv_ccw, local_sem):
        me = lax.axis_index("i")
        pos = _ring(me)
        right = _ring((pos + 1) % N_DEV)
        left = _ring((pos - 1) % N_DEV)

        barrier = pltpu.get_barrier_semaphore()
        for nbr in (left, right):
            pl.semaphore_signal(barrier, inc=1, device_id=(nbr,),
                                device_id_type=pl.DeviceIdType.MESH)
        pl.semaphore_wait(barrier, 2)

        def make_hop(h):
            o_send_cw = _ring((pos - h) % N_DEV)
            o_recv_cw = _ring((pos - 1 - h) % N_DEV)
            o_send_ccw = _ring((pos + h) % N_DEV)
            o_recv_ccw = _ring((pos + 1 + h) % N_DEV)
            src_cw = (x_ref.at[pl.ds(0, half)] if h == 0
                      else gx_cw.at[o_send_cw])
            src_ccw = (x_ref.at[pl.ds(half, half)] if h == 0
                       else gx_ccw.at[o_send_ccw])
            cw = pltpu.make_async_remote_copy(
                src_ref=src_cw, dst_ref=gx_cw.at[o_send_cw],
                send_sem=send_cw.at[h], recv_sem=recv_cw.at[h],
                device_id=(right,), device_id_type=pl.DeviceIdType.MESH)
            ccw = pltpu.make_async_remote_copy(
                src_ref=src_ccw, dst_ref=gx_ccw.at[o_send_ccw],
                send_sem=send_ccw.at[h], recv_sem=recv_ccw.at[h],
                device_id=(left,), device_id_type=pl.DeviceIdType.MESH)
            return cw, ccw, o_recv_cw, o_recv_ccw

        hops = [make_hop(h) for h in range(N_DEV - 1)]

        def gemm_rows(src_slice, out_row):
            cp = pltpu.make_async_copy(src_slice, stage, local_sem)
            cp.start()
            cp.wait()
            y = jnp.dot(stage[...], w_ref[...],
                        preferred_element_type=jnp.float32)
            out_ref[pl.ds(out_row, half), :] = _gelu(y)

        hops[0][0].start()
        hops[0][1].start()
        gemm_rows(x_ref.at[pl.ds(0, half)], me * m_per)
        gemm_rows(x_ref.at[pl.ds(half, half)], me * m_per + half)

        for h in range(N_DEV - 1):
            cw, ccw, o_recv_cw, o_recv_ccw = hops[h]
            cw.wait_recv()
            ccw.wait_recv()
            if h + 1 < N_DEV - 1:
                hops[h + 1][0].start()
                hops[h + 1][1].start()
            gemm_rows(gx_cw.at[o_recv_cw], o_recv_cw * m_per)
            gemm_rows(gx_ccw.at[o_recv_ccw], o_recv_ccw * m_per + half)

        for cw, ccw, _, _ in hops:
            cw.wait_send()
            ccw.wait_send()

    out, _, _ = pl.pallas_call(
        body,
        out_shape=[
            jax.ShapeDtypeStruct((m_glob, n_per), jnp.float32),
            jax.ShapeDtypeStruct((N_DEV, half, k), jnp.float32),
            jax.ShapeDtypeStruct((N_DEV, half, k), jnp.float32),
        ],
        in_specs=[
            pl.BlockSpec(memory_space=pltpu.MemorySpace.HBM),
            pl.BlockSpec(memory_space=pltpu.MemorySpace.VMEM),
        ],
        out_specs=[
            pl.BlockSpec(memory_space=pltpu.MemorySpace.VMEM),
            pl.BlockSpec(memory_space=pltpu.MemorySpace.HBM),
            pl.BlockSpec(memory_space=pltpu.MemorySpace.HBM),
        ],
        scratch_shapes=[
            pltpu.VMEM((half, k), jnp.float32),
            pltpu.SemaphoreType.DMA((N_DEV - 1,)),
            pltpu.SemaphoreType.DMA((N_DEV - 1,)),
            pltpu.SemaphoreType.DMA((N_DEV - 1,)),
            pltpu.SemaphoreType.DMA((N_DEV - 1,)),
            pltpu.SemaphoreType.DMA,
        ],
        compiler_params=pltpu.CompilerParams(
            collective_id=0,
            vmem_limit_bytes=64 * 1024 * 1024,
        ),
    )(x, w_mat)
    return out


# device time: 801447 ns/iter; 1.6556x vs baseline; 1.6556x over previous
import os
from pathlib import Path

import jax

try:
    jax.config.update("jax_compilation_cache_dir",
                      str(Path(__file__).parent / "jax_cache"))
except Exception:
    pass

import jax.numpy as jnp
from jax import lax
from jax.experimental import pallas as pl
from jax.experimental.pallas import tpu as pltpu

N_DEV = 8
_GELU_C = 0.7978845608028654


def _ring(q):
    return jnp.where(q < 4, q, 11 - q)


def _gelu(y):
    return 0.5 * y * (1.0 + jnp.tanh(_GELU_C * (y + 0.044715 * y * y * y)))


def kernel(x, w_mat):
    m_per, k = x.shape
    _, n_per = w_mat.shape
    kh = k // 2
    m_glob = N_DEV * m_per
    n_hops = N_DEV - 1

    def body(x_ref, w_ref, out_ref, gw_cw, gw_ccw, obox,
             stage, own_buf, send_buf,
             rs_cw, rr_cw, rs_ccw, rr_ccw, a2a_send, a2a_recv, lsem):
        me = lax.axis_index("i")
        pos = _ring(me)
        right = _ring((pos + 1) % N_DEV)
        left = _ring((pos - 1) % N_DEV)

        barrier = pltpu.get_barrier_semaphore()
        for d in range(1, N_DEV):
            pl.semaphore_signal(barrier, inc=1, device_id=((me + d) % N_DEV,),
                                device_id_type=pl.DeviceIdType.MESH)
        pl.semaphore_wait(barrier, N_DEV - 1)

        def make_hop(h):
            o_send_cw = _ring((pos - h) % N_DEV)
            o_recv_cw = _ring((pos - 1 - h) % N_DEV)
            o_send_ccw = _ring((pos + h) % N_DEV)
            o_recv_ccw = _ring((pos + 1 + h) % N_DEV)
            src_cw = w_ref.at[pl.ds(0, kh)] if h == 0 else gw_cw.at[o_send_cw]
            src_ccw = (w_ref.at[pl.ds(kh, kh)] if h == 0
                       else gw_ccw.at[o_send_ccw])
            cw = pltpu.make_async_remote_copy(
                src_ref=src_cw, dst_ref=gw_cw.at[o_send_cw],
                send_sem=rs_cw.at[h], recv_sem=rr_cw.at[h],
                device_id=(right,), device_id_type=pl.DeviceIdType.MESH)
            ccw = pltpu.make_async_remote_copy(
                src_ref=src_ccw, dst_ref=gw_ccw.at[o_send_ccw],
                send_sem=rs_ccw.at[h], recv_sem=rr_ccw.at[h],
                device_id=(left,), device_id_type=pl.DeviceIdType.MESH)
            return cw, ccw

        hops = [make_hop(h) for h in range(n_hops)]
        hops[0][0].start()
        hops[0][1].start()

        cp0 = pltpu.make_async_copy(w_ref.at[pl.ds(0, kh)],
                                    stage.at[pl.ds(0, kh)], lsem.at[0])
        cp1 = pltpu.make_async_copy(w_ref.at[pl.ds(kh, kh)],
                                    stage.at[pl.ds(kh, kh)], lsem.at[1])
        cp0.start()
        cp1.start()
        cp0.wait()
        cp1.wait()
        own_buf[...] = _gelu(jnp.dot(x_ref[...], stage[...],
                                     preferred_element_type=jnp.float32))
        cp_own = pltpu.make_async_copy(
            own_buf, out_ref.at[pl.ds(me * m_per, m_per)], lsem.at[2])
        cp_own.start()

        def process_origin(d):
            o = _ring((pos - d) % N_DEV)
            m = d - 1
            b = m % 2
            s0 = pltpu.make_async_copy(gw_cw.at[o], stage.at[pl.ds(0, kh)],
                                       lsem.at[0])
            s1 = pltpu.make_async_copy(gw_ccw.at[o], stage.at[pl.ds(kh, kh)],
                                       lsem.at[1])
            s0.start()
            s1.start()
            s0.wait()
            s1.wait()
            send_buf[b] = _gelu(jnp.dot(x_ref[...], stage[...],
                                        preferred_element_type=jnp.float32))
            ocp = pltpu.make_async_copy(send_buf.at[b], obox.at[m],
                                        lsem.at[3 + b])
            ocp.start()
            ocp.wait()
            d_send = pltpu.make_async_remote_copy(
                src_ref=obox.at[m],
                dst_ref=out_ref.at[pl.ds(me * m_per, m_per)],
                send_sem=a2a_send.at[m], recv_sem=a2a_recv.at[m],
                device_id=(o,), device_id_type=pl.DeviceIdType.MESH)
            d_send.start()
            return d_send

        sends = []
        for h in range(n_hops):
            cw, ccw = hops[h]
            cw.wait_recv()
            ccw.wait_recv()
            if h + 1 < n_hops:
                hops[h + 1][0].start()
                hops[h + 1][1].start()
            for d in {h + 1, n_hops - h}:
                if max(d - 1, n_hops - d) == h:
                    sends.append(process_origin(d))

        for cw, ccw in hops:
            cw.wait_send()
            ccw.wait_send()
        for snd in sends:
            snd.wait_send()
        cp_own.wait()
        for m in sorted(range(n_hops), key=lambda m: max(m, n_hops - 1 - m)):
            s = _ring((pos + m + 1) % N_DEV)
            rcv = pltpu.make_async_remote_copy(
                src_ref=obox.at[0],
                dst_ref=out_ref.at[pl.ds(s * m_per, m_per)],
                send_sem=a2a_send.at[0], recv_sem=a2a_recv.at[m],
                device_id=(me,), device_id_type=pl.DeviceIdType.MESH)
            rcv.wait_recv()

    out, _, _, _ = pl.pallas_call(
        body,
        out_shape=[
            jax.ShapeDtypeStruct((m_glob, n_per), jnp.float32),
            jax.ShapeDtypeStruct((N_DEV, kh, n_per), jnp.float32),
            jax.ShapeDtypeStruct((N_DEV, kh, n_per), jnp.float32),
            jax.ShapeDtypeStruct((n_hops, m_per, n_per), jnp.float32),
        ],
        in_specs=[
            pl.BlockSpec(memory_space=pltpu.MemorySpace.VMEM),
            pl.BlockSpec(memory_space=pltpu.MemorySpace.HBM),
        ],
        out_specs=[
            pl.BlockSpec(memory_space=pltpu.MemorySpace.HBM),
            pl.BlockSpec(memory_space=pltpu.MemorySpace.HBM),
            pl.BlockSpec(memory_space=pltpu.MemorySpace.HBM),
            pl.BlockSpec(memory_space=pltpu.MemorySpace.HBM),
        ],
        scratch_shapes=[
            pltpu.VMEM((k, n_per), jnp.float32),
            pltpu.VMEM((m_per, n_per), jnp.float32),
            pltpu.VMEM((2, m_per, n_per), jnp.float32),
            pltpu.SemaphoreType.DMA((n_hops,)),
            pltpu.SemaphoreType.DMA((n_hops,)),
            pltpu.SemaphoreType.DMA((n_hops,)),
            pltpu.SemaphoreType.DMA((n_hops,)),
            pltpu.SemaphoreType.DMA((n_hops,)),
            pltpu.SemaphoreType.DMA((n_hops,)),
            pltpu.SemaphoreType.DMA((5,)),
        ],
        compiler_params=pltpu.CompilerParams(
            collective_id=0,
            vmem_limit_bytes=64 * 1024 * 1024,
        ),
    )(x, w_mat)
    return out


# device time: 776313 ns/iter; 1.7092x vs baseline; 1.0324x over previous
from pathlib import Path

import jax

try:
    jax.config.update("jax_compilation_cache_dir",
                      str(Path(__file__).parent / "jax_cache"))
except Exception:
    pass

import jax.numpy as jnp
from jax import lax
from jax.experimental import pallas as pl
from jax.experimental.pallas import tpu as pltpu

N_DEV = 8
_GELU_C = 0.7978845608028654


def _ring(q):
    return jnp.where(q < 4, q, 11 - q)


def _gelu(y):
    return 0.5 * y * (1.0 + jnp.tanh(_GELU_C * (y + 0.044715 * y * y * y)))


def kernel(x, w_mat):
    m_per, k = x.shape
    _, n_per = w_mat.shape
    kh = k // 2
    m_glob = N_DEV * m_per
    n_hops = 4

    def body(x_ref, w_ref, out_ref, gcw_t, gcw_b, gccw_t, gccw_b, obox,
             stage, own_buf, send_buf,
             s_cwt, r_cwt, s_cwb, r_cwb, s_ccwt, r_ccwt, s_ccwb, r_ccwb,
             a2a_send, a2a_recv, lsem):
        me = lax.axis_index("i")
        pos = _ring(me)
        right = _ring((pos + 1) % N_DEV)
        left = _ring((pos - 1) % N_DEV)

        barrier = pltpu.get_barrier_semaphore()
        for d in range(1, N_DEV):
            pl.semaphore_signal(barrier, inc=1, device_id=((me + d) % N_DEV,),
                                device_id_type=pl.DeviceIdType.MESH)
        pl.semaphore_wait(barrier, N_DEV - 1)

        def rc(src, dst, ssem, rsem, dev):
            return pltpu.make_async_remote_copy(
                src_ref=src, dst_ref=dst, send_sem=ssem, recv_sem=rsem,
                device_id=(dev,), device_id_type=pl.DeviceIdType.MESH)

        def make_hop(h):
            o_cw = _ring((pos - h) % N_DEV)
            o_ccw = _ring((pos + h) % N_DEV)
            src_t = w_ref.at[pl.ds(0, kh)] if h == 0 else gcw_t.at[o_cw]
            src_b = w_ref.at[pl.ds(kh, kh)] if h == 0 else gcw_b.at[o_cw]
            src_t2 = w_ref.at[pl.ds(0, kh)] if h == 0 else gccw_t.at[o_ccw]
            src_b2 = w_ref.at[pl.ds(kh, kh)] if h == 0 else gccw_b.at[o_ccw]
            ds = [rc(src_t, gcw_t.at[o_cw], s_cwt.at[h], r_cwt.at[h], right)]
            if h <= 2:
                ds.append(rc(src_b, gcw_b.at[o_cw], s_cwb.at[h], r_cwb.at[h],
                             right))
                ds.append(rc(src_t2, gccw_t.at[o_ccw], s_ccwt.at[h],
                             r_ccwt.at[h], left))
            ds.append(rc(src_b2, gccw_b.at[o_ccw], s_ccwb.at[h],
                         r_ccwb.at[h], left))
            return ds

        hops = [make_hop(h) for h in range(n_hops)]
        for r in hops[0]:
            r.start()

        cp0 = pltpu.make_async_copy(w_ref.at[pl.ds(0, kh)],
                                    stage.at[pl.ds(0, kh)], lsem.at[0])
        cp1 = pltpu.make_async_copy(w_ref.at[pl.ds(kh, kh)],
                                    stage.at[pl.ds(kh, kh)], lsem.at[1])
        cp0.start()
        cp1.start()
        cp0.wait()
        cp1.wait()
        own_buf[...] = _gelu(jnp.dot(x_ref[...], stage[...],
                                     preferred_element_type=jnp.float32))
        cp_own = pltpu.make_async_copy(
            own_buf, out_ref.at[pl.ds(me * m_per, m_per)], lsem.at[2])
        cp_own.start()

        def process_origin(d):
            o = _ring((pos - d) % N_DEV)
            m = d - 1
            b = m % 2
            top = gcw_t if d <= 4 else gccw_t
            bot = gcw_b if d <= 3 else gccw_b
            s0 = pltpu.make_async_copy(top.at[o], stage.at[pl.ds(0, kh)],
                                       lsem.at[0])
            s1 = pltpu.make_async_copy(bot.at[o], stage.at[pl.ds(kh, kh)],
                                       lsem.at[1])
            s0.start()
            s1.start()
            s0.wait()
            s1.wait()
            send_buf[b] = _gelu(jnp.dot(x_ref[...], stage[...],
                                        preferred_element_type=jnp.float32))
            ocp = pltpu.make_async_copy(send_buf.at[b], obox.at[m],
                                        lsem.at[3 + b])
            ocp.start()
            ocp.wait()
            snd = rc(obox.at[m], out_ref.at[pl.ds(me * m_per, m_per)],
                     a2a_send.at[m], a2a_recv.at[m], o)
            snd.start()
            return snd

        sends = []
        for h in range(n_hops):
            for r in hops[h]:
                r.wait_recv()
            if h + 1 < n_hops:
                for r in hops[h + 1]:
                    r.start()
            if h <= 2:
                sends.append(process_origin(h + 1))
                sends.append(process_origin(7 - h))
            else:
                sends.append(process_origin(4))

        for hop in hops:
            for r in hop:
                r.wait_send()
        for snd in sends:
            snd.wait_send()
        cp_own.wait()
        for m in (0, 6, 1, 5, 2, 4, 3):
            s = _ring((pos + m + 1) % N_DEV)
            rcv = rc(obox.at[0], out_ref.at[pl.ds(s * m_per, m_per)],
                     a2a_send.at[0], a2a_recv.at[m], me)
            rcv.wait_recv()

    out = pl.pallas_call(
        body,
        out_shape=[
            jax.ShapeDtypeStruct((m_glob, n_per), jnp.float32),
            jax.ShapeDtypeStruct((N_DEV, kh, n_per), jnp.float32),
            jax.ShapeDtypeStruct((N_DEV, kh, n_per), jnp.float32),
            jax.ShapeDtypeStruct((N_DEV, kh, n_per), jnp.float32),
            jax.ShapeDtypeStruct((N_DEV, kh, n_per), jnp.float32),
            jax.ShapeDtypeStruct((N_DEV - 1, m_per, n_per), jnp.float32),
        ],
        in_specs=[
            pl.BlockSpec(memory_space=pltpu.MemorySpace.VMEM),
            pl.BlockSpec(memory_space=pltpu.MemorySpace.HBM),
        ],
        out_specs=[pl.BlockSpec(memory_space=pltpu.MemorySpace.HBM)] * 6,
        scratch_shapes=[
            pltpu.VMEM((k, n_per), jnp.float32),
            pltpu.VMEM((m_per, n_per), jnp.float32),
            pltpu.VMEM((2, m_per, n_per), jnp.float32),
            pltpu.SemaphoreType.DMA((4,)),
            pltpu.SemaphoreType.DMA((4,)),
            pltpu.SemaphoreType.DMA((4,)),
            pltpu.SemaphoreType.DMA((4,)),
            pltpu.SemaphoreType.DMA((4,)),
            pltpu.SemaphoreType.DMA((4,)),
            pltpu.SemaphoreType.DMA((4,)),
            pltpu.SemaphoreType.DMA((4,)),
            pltpu.SemaphoreType.DMA((N_DEV - 1,)),
            pltpu.SemaphoreType.DMA((N_DEV - 1,)),
            pltpu.SemaphoreType.DMA((5,)),
        ],
        compiler_params=pltpu.CompilerParams(
            collective_id=0,
            vmem_limit_bytes=64 * 1024 * 1024,
        ),
    )(x, w_mat)[0]
    return out
